# baseline (speedup 1.0000x reference)
import functools

import jax
import jax.numpy as jnp
from jax import lax
from jax.experimental import pallas as pl
from jax.experimental.pallas import tpu as pltpu
from jax.experimental.pallas import tpu_sc as plsc

_MIN_RES = 16
_MAX_RES = 1024
_HASH_PRIME = 19349663
_L = 16


def kernel(x, kernel):
    num_levels, T, F = kernel.shape
    b = jnp.exp((jnp.log(float(_MAX_RES)) - jnp.log(float(_MIN_RES)))
                / (num_levels - 1))
    n_levels = jnp.int32(jnp.floor(_MIN_RES * b ** jnp.arange(1, num_levels + 1)))
    table_flat = kernel.reshape(num_levels * T * F)
    x_rep = jnp.repeat(x, _L)
    n_dup = jnp.repeat(n_levels, F)

    mesh = plsc.VectorSubcoreMesh(core_axis_name="c", subcore_axis_name="s")

    @functools.partial(
        pl.kernel,
        mesh=mesh,
        out_type=jax.ShapeDtypeStruct((num_levels * F,), jnp.float32),
        scratch_types=[
            pltpu.VMEM((2 * _L,), jnp.float32),
            pltpu.VMEM((2 * _L,), jnp.int32),
            pltpu.VMEM((8 * _L,), jnp.int32),
            pltpu.VMEM((8 * _L,), jnp.float32),
            pltpu.VMEM((2 * _L,), jnp.float32),
            pltpu.SemaphoreType.DMA,
        ],
    )
    def sc_kernel(x_hbm, n_hbm, table_hbm, out_hbm,
                  x_v, n_v, idx_v, elems_v, out_v, sem):
        cid = lax.axis_index("c")
        sid = lax.axis_index("s")

        @pl.when(jnp.logical_and(cid == 0, sid == 0))
        def _():
            pltpu.sync_copy(x_hbm, x_v)
            pltpu.sync_copy(n_hbm, n_v)

            x0 = x_v[pl.ds(0, _L)]
            x1 = x_v[pl.ds(_L, _L)]
            fpar = jnp.arange(_L, dtype=jnp.int32) & 1
            jlane = jnp.arange(_L, dtype=jnp.int32) >> 1
            mask = jnp.int32(T - 1)
            prime = jnp.int32(_HASH_PRIME)

            weights = []
            for o in range(2):
                nd_f = n_v[pl.ds(o * _L, _L)].astype(jnp.float32)
                xl0 = (x0 * nd_f).astype(jnp.int32)
                xl1 = (x1 * nd_f).astype(jnp.int32)

                level = jlane + jnp.int32(8 * o)
                ebase = level * jnp.int32(T * F) + fpar
                hy0 = xl1 * prime
                hy1 = (xl1 + 1) * prime
                i00 = ebase + ((xl0 ^ hy0) & mask) * jnp.int32(F)
                i01 = ebase + ((xl0 ^ hy1) & mask) * jnp.int32(F)
                i10 = ebase + (((xl0 + 1) ^ hy0) & mask) * jnp.int32(F)
                i11 = ebase + (((xl0 + 1) ^ hy1) & mask) * jnp.int32(F)
                idx_v[pl.ds((0 + o) * _L, _L)] = i00
                idx_v[pl.ds((2 + o) * _L, _L)] = i01
                idx_v[pl.ds((4 + o) * _L, _L)] = i10
                idx_v[pl.ds((6 + o) * _L, _L)] = i11

                grid = 1.0 / nd_f
                vmin0 = xl0.astype(jnp.float32) * grid
                vmin1 = xl1.astype(jnp.float32) * grid
                vmax0 = vmin0 + grid
                vmax1 = vmin1 + grid
                d0 = vmax0 - vmin0
                d1 = vmax1 - vmin1
                weights.append(((vmax0 - x0) / d0, (x0 - vmin0) / d0,
                                (vmax1 - x1) / d1, (x1 - vmin1) / d1))

            pltpu.async_copy(table_hbm.at[idx_v], elems_v, sem).wait()

            for o in range(2):
                w1x, w2x, w1y, w2y = weights[o]
                f00 = elems_v[pl.ds((0 + o) * _L, _L)]
                f01 = elems_v[pl.ds((2 + o) * _L, _L)]
                f10 = elems_v[pl.ds((4 + o) * _L, _L)]
                f11 = elems_v[pl.ds((6 + o) * _L, _L)]
                out_v[pl.ds(o * _L, _L)] = (w1y * (w1x * f00 + w2x * f10)
                                            + w2y * (w1x * f01 + w2x * f11))

            pltpu.sync_copy(out_v, out_hbm)

    return sc_kernel(x_rep, n_dup, table_flat)


# speedup vs baseline: 521.0953x; 521.0953x over previous
import functools

import jax
import jax.numpy as jnp
from jax import lax
from jax.experimental import pallas as pl
from jax.experimental.pallas import tpu as pltpu
from jax.experimental.pallas import tpu_sc as plsc

_MIN_RES = 16
_MAX_RES = 1024
_HASH_PRIME = 19349663
_L = 16


def kernel(x, kernel):
    num_levels, T, F = kernel.shape
    b = jnp.exp((jnp.log(float(_MAX_RES)) - jnp.log(float(_MIN_RES)))
                / (num_levels - 1))
    n_levels = jnp.int32(jnp.floor(_MIN_RES * b ** jnp.arange(1, num_levels + 1)))
    table_flat = (kernel.reshape(num_levels, T // 128, 128, F)
                  .transpose(0, 1, 3, 2).reshape(-1))
    x_rep = jnp.repeat(x, _L)
    n_dup = jnp.repeat(n_levels, F)

    mesh = plsc.VectorSubcoreMesh(core_axis_name="c", subcore_axis_name="s")

    @functools.partial(
        pl.kernel,
        mesh=mesh,
        out_type=jax.ShapeDtypeStruct((num_levels * F,), jnp.float32),
        scratch_types=[
            pltpu.VMEM((2 * _L,), jnp.float32),
            pltpu.VMEM((2 * _L,), jnp.int32),
            pltpu.VMEM((8 * _L,), jnp.int32),
            pltpu.VMEM((8 * _L,), jnp.float32),
            pltpu.VMEM((2 * _L,), jnp.float32),
            pltpu.SemaphoreType.DMA,
        ],
    )
    def sc_kernel(x_hbm, n_hbm, table_hbm, out_hbm,
                  x_v, n_v, idx_v, elems_v, out_v, sem):
        cid = lax.axis_index("c")
        sid = lax.axis_index("s")

        @pl.when(jnp.logical_and(cid == 0, sid == 0))
        def _():
            pltpu.sync_copy(x_hbm, x_v)
            pltpu.sync_copy(n_hbm, n_v)

            x0 = x_v[pl.ds(0, _L)]
            x1 = x_v[pl.ds(_L, _L)]
            fpar = jnp.arange(_L, dtype=jnp.int32) & 1
            jlane = jnp.arange(_L, dtype=jnp.int32) >> 1
            mask = jnp.int32(T - 1)
            prime = jnp.int32(_HASH_PRIME)

            weights = []
            for o in range(2):
                nd_f = n_v[pl.ds(o * _L, _L)].astype(jnp.float32)
                xl0 = (x0 * nd_f).astype(jnp.int32)
                xl1 = (x1 * nd_f).astype(jnp.int32)

                level = jlane + jnp.int32(8 * o)
                ebase = level * jnp.int32(T * F) + fpar * jnp.int32(128)
                hy0 = xl1 * prime
                hy1 = (xl1 + 1) * prime
                h00 = (xl0 ^ hy0) & mask
                h01 = (xl0 ^ hy1) & mask
                h10 = ((xl0 + 1) ^ hy0) & mask
                h11 = ((xl0 + 1) ^ hy1) & mask
                i00 = ebase + ((h00 >> 7) << 8) + (h00 & 127)
                i01 = ebase + ((h01 >> 7) << 8) + (h01 & 127)
                i10 = ebase + ((h10 >> 7) << 8) + (h10 & 127)
                i11 = ebase + ((h11 >> 7) << 8) + (h11 & 127)
                idx_v[pl.ds((0 + o) * _L, _L)] = i00
                idx_v[pl.ds((2 + o) * _L, _L)] = i01
                idx_v[pl.ds((4 + o) * _L, _L)] = i10
                idx_v[pl.ds((6 + o) * _L, _L)] = i11

                grid = 1.0 / nd_f
                vmin0 = xl0.astype(jnp.float32) * grid
                vmin1 = xl1.astype(jnp.float32) * grid
                vmax0 = vmin0 + grid
                vmax1 = vmin1 + grid
                d0 = vmax0 - vmin0
                d1 = vmax1 - vmin1
                weights.append(((vmax0 - x0) / d0, (x0 - vmin0) / d0,
                                (vmax1 - x1) / d1, (x1 - vmin1) / d1))

            pltpu.async_copy(table_hbm.at[idx_v], elems_v, sem).wait()

            for o in range(2):
                w1x, w2x, w1y, w2y = weights[o]
                f00 = elems_v[pl.ds((0 + o) * _L, _L)]
                f01 = elems_v[pl.ds((2 + o) * _L, _L)]
                f10 = elems_v[pl.ds((4 + o) * _L, _L)]
                f11 = elems_v[pl.ds((6 + o) * _L, _L)]
                out_v[pl.ds(o * _L, _L)] = (w1y * (w1x * f00 + w2x * f10)
                                            + w2y * (w1x * f01 + w2x * f11))

            pltpu.sync_copy(out_v, out_hbm)

    return sc_kernel(x_rep, n_dup, table_flat)


# speedup vs baseline: 574.6582x; 1.1028x over previous
import functools

import jax
import jax.numpy as jnp
from jax import lax
from jax.experimental import pallas as pl
from jax.experimental.pallas import tpu as pltpu
from jax.experimental.pallas import tpu_sc as plsc

_HASH_PRIME = 19349663
_L = 16

_N_LEVELS = [21, 27, 36, 48, 64, 84, 111, 147, 194, 256,
             337, 445, 588, 776, 1024, 1351]


def kernel(x, kernel):
    num_levels, T, F = kernel.shape
    table_flat = (kernel.reshape(num_levels, T // 128, 128, F)
                  .transpose(0, 1, 3, 2).reshape(-1))

    mesh = plsc.VectorSubcoreMesh(core_axis_name="c", subcore_axis_name="s",
                                  num_cores=1, num_subcores=1)

    @functools.partial(
        pl.kernel,
        mesh=mesh,
        out_type=jax.ShapeDtypeStruct((num_levels * F,), jnp.float32),
        scratch_types=[
            pltpu.VMEM((_L,), jnp.float32),
            pltpu.VMEM((8 * _L,), jnp.int32),
            pltpu.VMEM((8 * _L,), jnp.float32),
            pltpu.VMEM((2 * _L,), jnp.float32),
            pltpu.SemaphoreType.DMA,
        ],
    )
    def sc_kernel(x_hbm, table_hbm, out_hbm, x_v, idx_v, elems_v, out_v, sem):
        pltpu.sync_copy(x_hbm, x_v.at[pl.ds(0, 2)])

        lane = jnp.arange(_L, dtype=jnp.int32)
        jlane = lane >> 1
        fpar = lane & 1
        xw = x_v[...]
        dn = lax.GatherDimensionNumbers(offset_dims=(), collapsed_slice_dims=(0,),
                                        start_index_map=(0,))
        x0 = lax.gather(xw, jnp.zeros((_L, 1), jnp.int32), dn, slice_sizes=(1,),
                        mode=lax.GatherScatterMode.PROMISE_IN_BOUNDS)
        x1 = lax.gather(xw, jnp.ones((_L, 1), jnp.int32), dn, slice_sizes=(1,),
                        mode=lax.GatherScatterMode.PROMISE_IN_BOUNDS)

        mask = jnp.int32(T - 1)
        prime = jnp.int32(_HASH_PRIME)

        weights = []
        for o in range(2):
            nd_f = jnp.full((_L,), float(_N_LEVELS[8 * o]), jnp.float32)
            for j in range(1, 8):
                nd_f = jnp.where(jlane == j,
                                 jnp.float32(_N_LEVELS[8 * o + j]), nd_f)
            ebase = (jlane + jnp.int32(8 * o)) * jnp.int32(T * F) + (fpar << 7)

            xl0 = (x0 * nd_f).astype(jnp.int32)
            xl1 = (x1 * nd_f).astype(jnp.int32)

            hy0 = xl1 * prime
            hy1 = (xl1 + 1) * prime
            h00 = (xl0 ^ hy0) & mask
            h01 = (xl0 ^ hy1) & mask
            h10 = ((xl0 + 1) ^ hy0) & mask
            h11 = ((xl0 + 1) ^ hy1) & mask
            idx_v[pl.ds((0 + o) * _L, _L)] = ebase + ((h00 >> 7) << 8) + (h00 & 127)
            idx_v[pl.ds((2 + o) * _L, _L)] = ebase + ((h01 >> 7) << 8) + (h01 & 127)
            idx_v[pl.ds((4 + o) * _L, _L)] = ebase + ((h10 >> 7) << 8) + (h10 & 127)
            idx_v[pl.ds((6 + o) * _L, _L)] = ebase + ((h11 >> 7) << 8) + (h11 & 127)

            grid = 1.0 / nd_f
            vmin0 = xl0.astype(jnp.float32) * grid
            vmin1 = xl1.astype(jnp.float32) * grid
            vmax0 = vmin0 + grid
            vmax1 = vmin1 + grid
            d0 = vmax0 - vmin0
            d1 = vmax1 - vmin1
            weights.append(((vmax0 - x0) / d0, (x0 - vmin0) / d0,
                            (vmax1 - x1) / d1, (x1 - vmin1) / d1))

        pltpu.async_copy(table_hbm.at[idx_v], elems_v, sem).wait()

        for o in range(2):
            w1x, w2x, w1y, w2y = weights[o]
            f00 = elems_v[pl.ds((0 + o) * _L, _L)]
            f01 = elems_v[pl.ds((2 + o) * _L, _L)]
            f10 = elems_v[pl.ds((4 + o) * _L, _L)]
            f11 = elems_v[pl.ds((6 + o) * _L, _L)]
            out_v[pl.ds(o * _L, _L)] = (w1y * (w1x * f00 + w2x * f10)
                                        + w2y * (w1x * f01 + w2x * f11))

        pltpu.sync_copy(out_v, out_hbm)

    return sc_kernel(x, table_flat)


# speedup vs baseline: 575.8798x; 1.0021x over previous
import functools

import jax
import jax.numpy as jnp
from jax import lax
from jax.experimental import pallas as pl
from jax.experimental.pallas import tpu as pltpu
from jax.experimental.pallas import tpu_sc as plsc

_HASH_PRIME = 19349663
_L = 16

_N_LEVELS = [21, 27, 36, 48, 64, 84, 111, 147, 194, 256,
             337, 445, 588, 776, 1024, 1351]


def kernel(x, kernel):
    num_levels, T, F = kernel.shape
    table_flat = (kernel.reshape(num_levels, T // 128, 128, F)
                  .transpose(0, 1, 3, 2).reshape(-1))

    mesh = plsc.VectorSubcoreMesh(core_axis_name="c", subcore_axis_name="s",
                                  num_cores=1, num_subcores=1)

    @functools.partial(
        pl.kernel,
        mesh=mesh,
        out_type=jax.ShapeDtypeStruct((num_levels * F,), jnp.float32),
        scratch_types=[
            pltpu.VMEM((_L,), jnp.float32),
            pltpu.VMEM((8 * _L,), jnp.int32),
            pltpu.VMEM((8 * _L,), jnp.float32),
            pltpu.VMEM((2 * _L,), jnp.float32),
            pltpu.SemaphoreType.DMA,
            pltpu.SemaphoreType.DMA,
        ],
    )
    def sc_kernel(x_hbm, table_hbm, out_hbm,
                  x_v, idx_v, elems_v, out_v, gsem, xsem):
        xcopy = pltpu.async_copy(x_hbm, x_v.at[pl.ds(0, 2)], xsem)

        lane = jnp.arange(_L, dtype=jnp.int32)
        jlane = lane >> 1
        fpar = lane & 1
        mask = jnp.int32(T - 1)
        prime = jnp.int32(_HASH_PRIME)
        nd_fs, ebases = [], []
        for o in range(2):
            nd_f = jnp.full((_L,), float(_N_LEVELS[8 * o]), jnp.float32)
            for j in range(1, 8):
                nd_f = jnp.where(jlane == j,
                                 jnp.float32(_N_LEVELS[8 * o + j]), nd_f)
            nd_fs.append(nd_f)
            ebases.append((jlane + jnp.int32(8 * o)) * jnp.int32(T * F)
                          + (fpar << 7))

        xcopy.wait()
        xw = x_v[...]
        dn = lax.GatherDimensionNumbers(offset_dims=(), collapsed_slice_dims=(0,),
                                        start_index_map=(0,))
        x0 = lax.gather(xw, jnp.zeros((_L, 1), jnp.int32), dn, slice_sizes=(1,),
                        mode=lax.GatherScatterMode.PROMISE_IN_BOUNDS)
        x1 = lax.gather(xw, jnp.ones((_L, 1), jnp.int32), dn, slice_sizes=(1,),
                        mode=lax.GatherScatterMode.PROMISE_IN_BOUNDS)

        xls = []
        for o in range(2):
            xl0 = (x0 * nd_fs[o]).astype(jnp.int32)
            xl1 = (x1 * nd_fs[o]).astype(jnp.int32)
            xls.append((xl0, xl1))

            hy0 = xl1 * prime
            hy1 = (xl1 + 1) * prime
            h00 = (xl0 ^ hy0) & mask
            h01 = (xl0 ^ hy1) & mask
            h10 = ((xl0 + 1) ^ hy0) & mask
            h11 = ((xl0 + 1) ^ hy1) & mask
            eb = ebases[o]
            idx_v[pl.ds((0 + o) * _L, _L)] = eb + ((h00 >> 7) << 8) + (h00 & 127)
            idx_v[pl.ds((2 + o) * _L, _L)] = eb + ((h01 >> 7) << 8) + (h01 & 127)
            idx_v[pl.ds((4 + o) * _L, _L)] = eb + ((h10 >> 7) << 8) + (h10 & 127)
            idx_v[pl.ds((6 + o) * _L, _L)] = eb + ((h11 >> 7) << 8) + (h11 & 127)

        gather = pltpu.async_copy(table_hbm.at[idx_v], elems_v, gsem)

        weights = []
        for o in range(2):
            xl0, xl1 = xls[o]
            grid = 1.0 / nd_fs[o]
            vmin0 = xl0.astype(jnp.float32) * grid
            vmin1 = xl1.astype(jnp.float32) * grid
            vmax0 = vmin0 + grid
            vmax1 = vmin1 + grid
            d0 = vmax0 - vmin0
            d1 = vmax1 - vmin1
            weights.append(((vmax0 - x0) / d0, (x0 - vmin0) / d0,
                            (vmax1 - x1) / d1, (x1 - vmin1) / d1))

        gather.wait()
        for o in range(2):
            w1x, w2x, w1y, w2y = weights[o]
            f00 = elems_v[pl.ds((0 + o) * _L, _L)]
            f01 = elems_v[pl.ds((2 + o) * _L, _L)]
            f10 = elems_v[pl.ds((4 + o) * _L, _L)]
            f11 = elems_v[pl.ds((6 + o) * _L, _L)]
            out_v[pl.ds(o * _L, _L)] = (w1y * (w1x * f00 + w2x * f10)
                                        + w2y * (w1x * f01 + w2x * f11))

        pltpu.sync_copy(out_v, out_hbm)

    return sc_kernel(x, table_flat)
